# baseline (speedup 1.0000x reference)
import functools

import jax
import jax.numpy as jnp
from jax import lax
from jax.experimental import pallas as pl
from jax.experimental.pallas import tpu as pltpu
from jax.experimental.pallas import tpu_sc as plsc

_NC = 2
_NS = 16
_L = 16

_C = 8

_N0 = 72
_N1 = 8
_NB0 = 3
_NB1 = 2


def _build_kernel(B_pad, S, D, scale, n0, n1):
    C = _C
    total_chunks = B_pad // C
    assert _NS * (n0 + n1) == total_chunks
    n_max = max(n0, n1)
    mesh = plsc.VectorSubcoreMesh(core_axis_name="c", subcore_axis_name="s")

    @functools.partial(
        pl.kernel,
        out_type=jax.ShapeDtypeStruct((B_pad, D), jnp.float32),
        mesh=mesh,
        compiler_params=pltpu.CompilerParams(needs_layout_passes=False),
        scratch_types=[
            pltpu.VMEM((n_max, C * S), jnp.int32),
            pltpu.VMEM((C * S, D // 2), jnp.float32),
            pltpu.VMEM((C * S, D // 2), jnp.float32),
            pltpu.VMEM((C * S, D // 2), jnp.float32),
            pltpu.VMEM((C, D), jnp.float32),
            pltpu.VMEM((C, D), jnp.float32),
            pltpu.VMEM((C, D), jnp.float32),
            pltpu.SemaphoreType.DMA,
            pltpu.SemaphoreType.DMA,
            pltpu.SemaphoreType.DMA,
            pltpu.SemaphoreType.DMA,
            pltpu.SemaphoreType.DMA,
            pltpu.SemaphoreType.DMA,
        ],
    )
    def aggr(feat_hbm, nidx_hbm, out_hbm, idx_ref, g0, g1, g2, o0, o1, o2,
             sg0, sg1, sg2, so0, so1, so2):
        cid = lax.axis_index("c")
        sid = lax.axis_index("s")
        gbufs = ((g0, sg0), (g1, sg1), (g2, sg2))
        obufs = ((o0, so0), (o1, so1), (o2, so2))

        def run(n_chunks, start_chunk, nb):
            assert n_chunks % nb == 0 and n_chunks // nb >= 2
            pltpu.sync_copy(nidx_hbm.at[pl.ds(start_chunk, n_chunks)],
                            idx_ref.at[pl.ds(0, n_chunks)])
            base_row = start_chunk * C

            for b in range(nb):
                g, sg = gbufs[b]
                pltpu.async_copy(feat_hbm.at[idx_ref.at[b]], g, sg)

            def group_body(p, carry):
                j = p * nb
                for b in range(nb):
                    g, sg = gbufs[b]
                    o, so = obufs[b]
                    jj = j + b
                    pltpu.make_async_copy(
                        feat_hbm.at[idx_ref.at[jj]], g, sg).wait()

                    @pl.when(jj >= nb)
                    def _wait_out():
                        pltpu.make_async_copy(
                            o, out_hbm.at[pl.ds(base_row + (jj - nb) * C, C)],
                            so).wait()

                    def reduce_row(r, c2):
                        row = r * S
                        for wg in range(D // 32):
                            sl = pl.ds(wg * _L, _L)
                            ab0 = plsc.bitcast(g[row, sl], jnp.bfloat16)
                            a0, b0 = plsc.unpack(
                                ab0, format=plsc.PackFormat.INTERLEAVED)
                            acc_a, acc_b = a0, b0
                            for s in range(1, S):
                                ab = plsc.bitcast(g[row + s, sl],
                                                  jnp.bfloat16)
                                a, b = plsc.unpack(
                                    ab, format=plsc.PackFormat.INTERLEAVED)
                                acc_a = acc_a + a
                                acc_b = acc_b + b
                            o[r, pl.ds(wg * 32, _L)] = acc_a * scale
                            o[r, pl.ds(wg * 32 + _L, _L)] = acc_b * scale
                        return c2

                    lax.fori_loop(0, C, reduce_row, 0, unroll=False)
                    pltpu.async_copy(
                        o, out_hbm.at[pl.ds(base_row + jj * C, C)], so)

                    @pl.when(jj + nb < n_chunks)
                    def _next_gather():
                        pltpu.async_copy(
                            feat_hbm.at[idx_ref.at[jj + nb]], g, sg)

                return carry

            lax.fori_loop(0, n_chunks // nb, group_body, 0, unroll=False)
            for b in range(nb):
                o, so = obufs[b]
                pltpu.make_async_copy(
                    o, out_hbm.at[pl.ds(base_row + (n_chunks - nb + b) * C, C)],
                    so).wait()

        @pl.when(cid == 0)
        def _core0():
            run(n0, sid * n0, _NB0)

        if n1 > 0:
            @pl.when(cid == 1)
            def _core1():
                run(n1, _NS * n0 + sid * n1, _NB1)

    return aggr


def kernel(features, nodes, neigh_idx, num_sample):
    N, D = features.shape
    B, S = neigh_idx.shape
    B_pad = _C * _NS * (_N0 + _N1)
    assert B_pad >= B
    nidx = neigh_idx.astype(jnp.int32)
    if B_pad != B:
        nidx = jnp.pad(nidx, ((0, B_pad - B), (0, 0)))
    nidx = nidx.reshape(B_pad // _C, _C * S)

    featp = features.astype(jnp.bfloat16).reshape(N, D // 32, 2, _L)
    featp = featp.transpose(0, 1, 3, 2).reshape(N, D // 2, 2)
    featp = jax.lax.bitcast_convert_type(featp, jnp.float32)

    aggr = _build_kernel(B_pad, S, D, 1.0 / float(S), _N0, _N1)
    out = aggr(featp, nidx)
    return out[:B]


# speedup vs baseline: 1.4086x; 1.4086x over previous
import functools

import jax
import jax.numpy as jnp
from jax import lax
from jax.experimental import pallas as pl
from jax.experimental.pallas import tpu as pltpu
from jax.experimental.pallas import tpu_sc as plsc

_NC = 2
_NS = 16
_L = 16

_C = 8

_N0 = 72
_N1 = 8
_NB0 = 2
_NB1 = 2


def _build_kernel(B_pad, S, D, scale, n0, n1):
    C = _C
    total_chunks = B_pad // C
    assert _NS * (n0 + n1) == total_chunks
    n_max = max(n0, n1)
    mesh = plsc.VectorSubcoreMesh(core_axis_name="c", subcore_axis_name="s")

    @functools.partial(
        pl.kernel,
        out_type=jax.ShapeDtypeStruct((B_pad, D), jnp.float32),
        mesh=mesh,
        scratch_types=[
            pltpu.VMEM((n_max, C * S), jnp.int32),
            pltpu.VMEM((C * S, D), jnp.float32),
            pltpu.VMEM((C * S, D), jnp.float32),
            pltpu.VMEM((C * S, D), jnp.float32),
            pltpu.VMEM((C, D), jnp.float32),
            pltpu.VMEM((C, D), jnp.float32),
            pltpu.VMEM((C, D), jnp.float32),
            pltpu.SemaphoreType.DMA,
            pltpu.SemaphoreType.DMA,
            pltpu.SemaphoreType.DMA,
            pltpu.SemaphoreType.DMA,
            pltpu.SemaphoreType.DMA,
            pltpu.SemaphoreType.DMA,
        ],
    )
    def aggr(feat_hbm, nidx_hbm, out_hbm, idx_ref, g0, g1, g2, o0, o1, o2,
             sg0, sg1, sg2, so0, so1, so2):
        cid = lax.axis_index("c")
        sid = lax.axis_index("s")
        gbufs = ((g0, sg0), (g1, sg1), (g2, sg2))
        obufs = ((o0, so0), (o1, so1), (o2, so2))

        def run(n_chunks, start_chunk, nb):
            assert n_chunks % nb == 0 and n_chunks // nb >= 2
            pltpu.sync_copy(nidx_hbm.at[pl.ds(start_chunk, n_chunks)],
                            idx_ref.at[pl.ds(0, n_chunks)])
            base_row = start_chunk * C

            for b in range(nb):
                g, sg = gbufs[b]
                pltpu.async_copy(feat_hbm.at[idx_ref.at[b]], g, sg)

            def group_body(p, carry):
                j = p * nb
                for b in range(nb):
                    g, sg = gbufs[b]
                    o, so = obufs[b]
                    jj = j + b
                    pltpu.make_async_copy(
                        feat_hbm.at[idx_ref.at[jj]], g, sg).wait()

                    @pl.when(jj >= nb)
                    def _wait_out():
                        pltpu.make_async_copy(
                            o, out_hbm.at[pl.ds(base_row + (jj - nb) * C, C)],
                            so).wait()

                    def reduce_row(r, c2):
                        row = r * S
                        for v in range(D // _L):
                            sl = pl.ds(v * _L, _L)
                            acc = g[row, sl]
                            for s in range(1, S):
                                acc = acc + g[row + s, sl]
                            o[r, sl] = acc * scale
                        return c2

                    lax.fori_loop(0, C, reduce_row, 0, unroll=False)
                    pltpu.async_copy(
                        o, out_hbm.at[pl.ds(base_row + jj * C, C)], so)

                    @pl.when(jj + nb < n_chunks)
                    def _next_gather():
                        pltpu.async_copy(
                            feat_hbm.at[idx_ref.at[jj + nb]], g, sg)

                return carry

            lax.fori_loop(0, n_chunks // nb, group_body, 0, unroll=False)
            for b in range(nb):
                o, so = obufs[b]
                pltpu.make_async_copy(
                    o, out_hbm.at[pl.ds(base_row + (n_chunks - nb + b) * C, C)],
                    so).wait()

        @pl.when(cid == 0)
        def _core0():
            run(n0, sid * n0, _NB0)

        if n1 > 0:
            @pl.when(cid == 1)
            def _core1():
                run(n1, _NS * n0 + sid * n1, _NB1)

    return aggr


def kernel(features, nodes, neigh_idx, num_sample):
    N, D = features.shape
    B, S = neigh_idx.shape
    B_pad = _C * _NS * (_N0 + _N1)
    assert B_pad >= B
    nidx = neigh_idx.astype(jnp.int32)
    if B_pad != B:
        nidx = jnp.pad(nidx, ((0, B_pad - B), (0, 0)))
    nidx = nidx.reshape(B_pad // _C, _C * S)

    aggr = _build_kernel(B_pad, S, D, 1.0 / float(S), _N0, _N1)
    out = aggr(features, nidx)
    return out[:B]
